# baseline (speedup 1.0000x reference)
import functools

import jax
import jax.numpy as jnp
from jax import lax
from jax.experimental import pallas as pl
from jax.experimental.pallas import tpu as pltpu
from jax.experimental.pallas import tpu_sc as plsc

N = 10000
E = 160000
D = 256
NCLS = 128

NC = 2
NS = 16
B = 128
E_PAD = 163840
NBLK_TILE = E_PAD // (NS * B)
NBLK_DEG = E_PAD // (NC * NS * B)
RPT = N // NS


def _zero16():
  return jnp.zeros((16,), jnp.float32)


def _bcast_lane(wv, l):
  idx = jnp.full((16, 1), l, jnp.int32)
  dn = lax.GatherDimensionNumbers(
      offset_dims=(), collapsed_slice_dims=(0,), start_index_map=(0,))
  return lax.gather(wv, idx, dn, slice_sizes=(1,),
                    mode=lax.GatherScatterMode.PROMISE_IN_BOUNDS)


def _deg_body(c_hbm, ew_hbm, deg0_hbm, deg1_hbm, c_v, ew_v, zv, deg_sh):
  cid = lax.axis_index("c")
  sid = lax.axis_index("s")
  wid = cid * NS + sid

  z = _zero16()
  for i in range(40):
    zv[pl.ds(i * 16, 16)] = z

  @pl.when(sid < 15)
  def _():
    pltpu.sync_copy(zv, deg_sh.at[pl.ds(sid * 640, 640)])

  @pl.when(sid == 15)
  def _():
    pltpu.sync_copy(zv.at[pl.ds(0, 400)], deg_sh.at[pl.ds(9600, 400)])

  plsc.subcore_barrier()

  base = wid * NBLK_DEG
  pltpu.sync_copy(c_hbm.at[pl.ds(base, NBLK_DEG)], c_v)
  pltpu.sync_copy(ew_hbm.at[pl.ds(base, NBLK_DEG)], ew_v)

  def blk(b, carry):
    pltpu.sync_copy(ew_v.at[b], deg_sh.at[c_v.at[b]], add=True)
    return carry

  lax.fori_loop(0, NBLK_DEG, blk, 0)
  plsc.subcore_barrier()

  def wb(dst):
    @pl.when(sid < 15)
    def _():
      pltpu.sync_copy(deg_sh.at[pl.ds(sid * 640, 640)], zv)
      pltpu.sync_copy(zv, dst.at[pl.ds(sid * 640, 640)])

    @pl.when(sid == 15)
    def _():
      pltpu.sync_copy(deg_sh.at[pl.ds(9600, 400)], zv.at[pl.ds(0, 400)])
      pltpu.sync_copy(zv.at[pl.ds(0, 400)], dst.at[pl.ds(9600, 400)])

  @pl.when(cid == 0)
  def _():
    wb(deg0_hbm)

  @pl.when(cid == 1)
  def _():
    wb(deg1_hbm)


def _make_deg_call(mesh):
  return pl.kernel(
      _deg_body,
      out_type=(
          jax.ShapeDtypeStruct((N,), jnp.float32),
          jax.ShapeDtypeStruct((N,), jnp.float32),
      ),
      mesh=mesh,
      scratch_types=[
          pltpu.VMEM((NBLK_DEG, B), jnp.int32),
          pltpu.VMEM((NBLK_DEG, B), jnp.float32),
          pltpu.VMEM((640,), jnp.float32),
          pltpu.VMEM_SHARED((N,), jnp.float32),
      ],
  )


def _prop_body(wh, g_lo, g_hi, r_hbm, c_hbm, ew_hbm, out_lo, out_hi,
               c_all, rv0, rv1, ev0, ev1, rows0, rows1,
               gs0, gs1, ss0, ss1, ms0, ms1, acc_sh):
  cid = lax.axis_index("c")
  sid = lax.axis_index("s")
  nseg = wh // 16

  _acc_init(acc_sh, rows0, sid, nseg)
  plsc.subcore_barrier()

  pltpu.sync_copy(c_hbm.at[pl.ds(sid * NBLK_TILE, NBLK_TILE)], c_all)

  def run(g_ref):
    _pipelined_edges(g_ref, acc_sh, r_hbm, ew_hbm, c_all, sid * NBLK_TILE,
                     rv0, rv1, ev0, ev1, rows0, rows1,
                     gs0, gs1, ss0, ss1, ms0, ms1, NBLK_TILE, nseg)

  @pl.when(cid == 0)
  def _():
    run(g_lo)

  @pl.when(cid == 1)
  def _():
    run(g_hi)

  plsc.subcore_barrier()

  @pl.when(cid == 0)
  def _():
    _acc_writeback(acc_sh, out_lo, sid)

  @pl.when(cid == 1)
  def _():
    _acc_writeback(acc_sh, out_hi, sid)


def _acc_init(acc_sh, rows_v, sid, nseg):
  z = _zero16()

  def zr(j, carry):
    for k in range(nseg):
      rows_v[j, pl.ds(k * 16, 16)] = z
    return carry

  lax.fori_loop(0, B, zr, 0)
  base_row = sid * 640

  @pl.when(sid < 15)
  def _():
    for kk in range(5):
      pltpu.sync_copy(rows_v, acc_sh.at[pl.ds(base_row + kk * B, B)])

  @pl.when(sid == 15)
  def _():
    for kk in range(3):
      pltpu.sync_copy(rows_v, acc_sh.at[pl.ds(base_row + kk * B, B)])
    pltpu.sync_copy(rows_v.at[pl.ds(0, 16)], acc_sh.at[pl.ds(9984, 16)])


def _acc_writeback(acc_sh, dst, sid):
  base_row = sid * 640

  @pl.when(sid < 15)
  def _():
    pltpu.sync_copy(acc_sh.at[pl.ds(base_row, 640)],
                    dst.at[pl.ds(base_row, 640)])

  @pl.when(sid == 15)
  def _():
    pltpu.sync_copy(acc_sh.at[pl.ds(base_row, 400)],
                    dst.at[pl.ds(base_row, 400)])


def _scale_block(rows_ref, ev, nseg):
  @plsc.parallel_loop(0, B // 16, 1, unroll=2)
  def _(jj):
    wv = ev[pl.ds(jj * 16, 16)]
    for l in range(16):
      j = jj * 16 + l
      w = _bcast_lane(wv, l)
      for k in range(nseg):
        rows_ref[j, pl.ds(k * 16, 16)] = rows_ref[j, pl.ds(k * 16, 16)] * w


def _pipelined_edges(g_ref, acc_sh, r_hbm, ew_hbm, c_all, blkbase,
                     rv0, rv1, ev0, ev1, rows0, rows1,
                     gs0, gs1, ss0, ss1, ms0, ms1, nblk, nseg):
  def meta_start(b, rv, ev, ms):
    pltpu.async_copy(r_hbm.at[blkbase + b], rv, ms)
    pltpu.async_copy(ew_hbm.at[pl.ds((blkbase + b) * B, B)], ev, ms)

  def meta_wait(b, rv, ev, ms):
    pltpu.make_async_copy(r_hbm.at[blkbase + b], rv, ms).wait()
    pltpu.make_async_copy(ew_hbm.at[pl.ds((blkbase + b) * B, B)], ev, ms).wait()

  pltpu.sync_copy(r_hbm.at[blkbase], rv0)
  pltpu.sync_copy(ew_hbm.at[pl.ds(blkbase * B, B)], ev0)
  pltpu.async_copy(g_ref.at[rv0], rows0, gs0)
  meta_start(1, rv1, ev1, ms1)

  npair = nblk // 2

  def do_pair(p, first, last):
    b0 = 2 * p
    b1 = b0 + 1
    pltpu.make_async_copy(g_ref.at[rv0], rows0, gs0).wait()
    if not first:
      pltpu.make_async_copy(rows1, acc_sh.at[c_all.at[b0 - 1]], ss1).wait()
    meta_wait(b1, rv1, ev1, ms1)
    pltpu.async_copy(g_ref.at[rv1], rows1, gs1)
    _scale_block(rows0, ev0, nseg)
    sd0 = pltpu.async_copy(rows0, acc_sh.at[c_all.at[b0]], ss0, add=True)
    if not last:
      meta_start(b0 + 2, rv0, ev0, ms0)
    pltpu.make_async_copy(g_ref.at[rv1], rows1, gs1).wait()
    sd0.wait()
    if not last:
      meta_wait(b0 + 2, rv0, ev0, ms0)
      pltpu.async_copy(g_ref.at[rv0], rows0, gs0)
    _scale_block(rows1, ev1, nseg)
    pltpu.async_copy(rows1, acc_sh.at[c_all.at[b1]], ss1, add=True)
    if not last:
      meta_start(b1 + 2, rv1, ev1, ms1)

  def pair(p, carry):
    do_pair(p, False, False)
    return carry

  do_pair(0, True, npair == 1)
  lax.fori_loop(1, npair - 1, pair, 0)
  do_pair(npair - 1, False, True)
  pltpu.make_async_copy(rows1, acc_sh.at[c_all.at[nblk - 1]], ss1).wait()


def _prop_split_body(g_hbm, r_hbm, c_hbm, ew_hbm, out0, out1,
                     c_all, rv0, rv1, ev0, ev1, rows0, rows1,
                     gs0, gs1, ss0, ss1, ms0, ms1, acc_sh):
  cid = lax.axis_index("c")
  sid = lax.axis_index("s")
  nseg = NCLS // 16
  nblk = NBLK_DEG

  _acc_init(acc_sh, rows0, sid, nseg)
  plsc.subcore_barrier()

  wid = cid * NS + sid
  pltpu.sync_copy(c_hbm.at[pl.ds(wid * nblk, nblk)], c_all)

  _pipelined_edges(g_hbm, acc_sh, r_hbm, ew_hbm, c_all, wid * nblk,
                   rv0, rv1, ev0, ev1, rows0, rows1,
                   gs0, gs1, ss0, ss1, ms0, ms1, nblk, nseg)
  plsc.subcore_barrier()

  @pl.when(cid == 0)
  def _():
    _acc_writeback(acc_sh, out0, sid)

  @pl.when(cid == 1)
  def _():
    _acc_writeback(acc_sh, out1, sid)


def _make_prop_split_call(mesh):
  return pl.kernel(
      _prop_split_body,
      out_type=(
          jax.ShapeDtypeStruct((N, NCLS), jnp.float32),
          jax.ShapeDtypeStruct((N, NCLS), jnp.float32),
      ),
      mesh=mesh,
      scratch_types=[
          pltpu.VMEM((NBLK_DEG, B), jnp.int32),
          pltpu.VMEM((B,), jnp.int32),
          pltpu.VMEM((B,), jnp.int32),
          pltpu.VMEM((B,), jnp.float32),
          pltpu.VMEM((B,), jnp.float32),
          pltpu.VMEM((B, NCLS), jnp.float32),
          pltpu.VMEM((B, NCLS), jnp.float32),
          pltpu.SemaphoreType.DMA,
          pltpu.SemaphoreType.DMA,
          pltpu.SemaphoreType.DMA,
          pltpu.SemaphoreType.DMA,
          pltpu.SemaphoreType.DMA,
          pltpu.SemaphoreType.DMA,
          pltpu.VMEM_SHARED((N, NCLS), jnp.float32),
      ],
  )


def _make_prop_call(mesh, wh):
  return pl.kernel(
      functools.partial(_prop_body, wh),
      out_type=(
          jax.ShapeDtypeStruct((N, wh), jnp.float32),
          jax.ShapeDtypeStruct((N, wh), jnp.float32),
      ),
      mesh=mesh,
      scratch_types=[
          pltpu.VMEM((NBLK_TILE, B), jnp.int32),
          pltpu.VMEM((B,), jnp.int32),
          pltpu.VMEM((B,), jnp.int32),
          pltpu.VMEM((B,), jnp.float32),
          pltpu.VMEM((B,), jnp.float32),
          pltpu.VMEM((B, wh), jnp.float32),
          pltpu.VMEM((B, wh), jnp.float32),
          pltpu.SemaphoreType.DMA,
          pltpu.SemaphoreType.DMA,
          pltpu.SemaphoreType.DMA,
          pltpu.SemaphoreType.DMA,
          pltpu.SemaphoreType.DMA,
          pltpu.SemaphoreType.DMA,
          pltpu.VMEM_SHARED((N, wh), jnp.float32),
      ],
  )


_GRID = 5
_RB = N // _GRID


def _row_spec(cols):
  return pl.BlockSpec((_RB, cols), lambda i: (i, 0))


def _full_spec(r, cols):
  return pl.BlockSpec((r, cols), lambda i: (0, 0))


def _dis(deg0_ref, deg1_ref):
  return lax.rsqrt(deg0_ref[...] + deg1_ref[...] + 1.0)


def _k1_body(x_ref, py_ref, lw_ref, lb_ref, w1_ref, d0_ref, d1_ref,
             glo_ref, ghi_ref):
  t = x_ref[...] + jnp.dot(py_ref[...], lw_ref[...],
                           preferred_element_type=jnp.float32) + lb_ref[...]
  g = _dis(d0_ref, d1_ref) * jnp.dot(t, w1_ref[...],
                                     preferred_element_type=jnp.float32)
  glo_ref[...] = g[:, :128]
  ghi_ref[...] = g[:, 128:]


def _k2_body(alo_ref, ahi_ref, glo_ref, ghi_ref, d0_ref, d1_ref, b_ref,
             w_ref, olo_ref, ohi_ref):
  dis = _dis(d0_ref, d1_ref)
  acc = jnp.concatenate([alo_ref[...], ahi_ref[...]], axis=1)
  g = jnp.concatenate([glo_ref[...], ghi_ref[...]], axis=1)
  h = jax.nn.relu(dis * (acc + g) + b_ref[...])
  g2 = dis * jnp.dot(h, w_ref[...], preferred_element_type=jnp.float32)
  olo_ref[...] = g2[:, :128]
  ohi_ref[...] = g2[:, 128:]


def _k3_body(alo_ref, ahi_ref, glo_ref, ghi_ref, d0_ref, d1_ref, b_ref,
             w_ref, f_ref, g3_ref):
  dis = _dis(d0_ref, d1_ref)
  acc = jnp.concatenate([alo_ref[...], ahi_ref[...]], axis=1)
  g = jnp.concatenate([glo_ref[...], ghi_ref[...]], axis=1)
  f = dis * (acc + g) + b_ref[...]
  f_ref[...] = f
  h2 = jax.nn.relu(f)
  g3_ref[...] = dis * jnp.dot(h2, w_ref[...],
                              preferred_element_type=jnp.float32)


def _k4_body(a0_ref, a1_ref, g_ref, d0_ref, d1_ref, b_ref, out_ref):
  dis = _dis(d0_ref, d1_ref)
  acc = a0_ref[...] + a1_ref[...]
  out_ref[...] = dis * (acc + g_ref[...]) + b_ref[...]


def kernel(x, pseudo_y, edge_index, egde_values, lin_W, lin_b,
           W1, b1, W2, b2, W3, b3):
  i32 = jnp.int32
  r = edge_index[0].astype(i32)
  c = edge_index[1].astype(i32)
  ew = egde_values.astype(jnp.float32)

  pad = E_PAD - E
  fill = (jnp.arange(pad, dtype=i32) * 13) % N
  r_p = jnp.concatenate([r, fill])
  c_p = jnp.concatenate([c, fill])
  ew_p = jnp.concatenate([ew, jnp.zeros((pad,), jnp.float32)])
  r2d = r_p.reshape(E_PAD // B, B)
  c2d = c_p.reshape(E_PAD // B, B)

  mesh = plsc.VectorSubcoreMesh(core_axis_name="c", subcore_axis_name="s")

  deg0, deg1 = _make_deg_call(mesh)(c2d, ew_p.reshape(E_PAD // B, B))
  d0 = deg0.reshape(N, 1)
  d1 = deg1.reshape(N, 1)

  dspec = pl.BlockSpec((_RB, 1), lambda i: (i, 0))
  lb2 = lin_b.reshape(1, D)
  b1_2 = b1.reshape(1, D)
  b2_2 = b2.reshape(1, D)
  b3_2 = b3.reshape(1, NCLS)

  g1_lo, g1_hi = pl.pallas_call(
      _k1_body,
      grid=(_GRID,),
      in_specs=[_row_spec(D), _row_spec(10), _full_spec(10, D),
                _full_spec(1, D), _full_spec(D, D), dspec, dspec],
      out_specs=(_row_spec(128), _row_spec(128)),
      out_shape=(jax.ShapeDtypeStruct((N, 128), jnp.float32),
                 jax.ShapeDtypeStruct((N, 128), jnp.float32)),
  )(x, pseudo_y, lin_W, lb2, W1, d0, d1)

  prop256 = _make_prop_call(mesh, 128)
  acc1_lo, acc1_hi = prop256(g1_lo, g1_hi, r2d, c2d, ew_p)

  g2_lo, g2_hi = pl.pallas_call(
      _k2_body,
      grid=(_GRID,),
      in_specs=[_row_spec(128), _row_spec(128), _row_spec(128), _row_spec(128),
                dspec, dspec, _full_spec(1, D), _full_spec(D, D)],
      out_specs=(_row_spec(128), _row_spec(128)),
      out_shape=(jax.ShapeDtypeStruct((N, 128), jnp.float32),
                 jax.ShapeDtypeStruct((N, 128), jnp.float32)),
  )(acc1_lo, acc1_hi, g1_lo, g1_hi, d0, d1, b1_2, W2)

  acc2_lo, acc2_hi = prop256(g2_lo, g2_hi, r2d, c2d, ew_p)

  f, g3 = pl.pallas_call(
      _k3_body,
      grid=(_GRID,),
      in_specs=[_row_spec(128), _row_spec(128), _row_spec(128), _row_spec(128),
                dspec, dspec, _full_spec(1, D), _full_spec(D, NCLS)],
      out_specs=(_row_spec(D), _row_spec(NCLS)),
      out_shape=(jax.ShapeDtypeStruct((N, D), jnp.float32),
                 jax.ShapeDtypeStruct((N, NCLS), jnp.float32)),
  )(acc2_lo, acc2_hi, g2_lo, g2_hi, d0, d1, b2_2, W3)

  prop3 = _make_prop_split_call(mesh)
  acc3_0, acc3_1 = prop3(g3, r2d, c2d, ew_p)

  out = pl.pallas_call(
      _k4_body,
      grid=(_GRID,),
      in_specs=[_row_spec(NCLS), _row_spec(NCLS), _row_spec(NCLS),
                dspec, dspec, _full_spec(1, NCLS)],
      out_specs=_row_spec(NCLS),
      out_shape=jax.ShapeDtypeStruct((N, NCLS), jnp.float32),
  )(acc3_0, acc3_1, g3, d0, d1, b3_2)

  return (f, out)


# speedup vs baseline: 1.0163x; 1.0163x over previous
import functools

import jax
import jax.numpy as jnp
from jax import lax
from jax.experimental import pallas as pl
from jax.experimental.pallas import tpu as pltpu
from jax.experimental.pallas import tpu_sc as plsc

N = 10000
E = 160000
D = 256
NCLS = 128

NC = 2
NS = 16
B = 128
E_PAD = 163840
NBLK_TILE = E_PAD // (NS * B)
NBLK_DEG = E_PAD // (NC * NS * B)
RPT = N // NS


def _zero16():
  return jnp.zeros((16,), jnp.float32)


def _bcast_lane(wv, l):
  idx = jnp.full((16, 1), l, jnp.int32)
  dn = lax.GatherDimensionNumbers(
      offset_dims=(), collapsed_slice_dims=(0,), start_index_map=(0,))
  return lax.gather(wv, idx, dn, slice_sizes=(1,),
                    mode=lax.GatherScatterMode.PROMISE_IN_BOUNDS)


def _deg_body(c_hbm, ew_hbm, deg0_hbm, deg1_hbm, c_v, ew_v, zv, deg_sh):
  cid = lax.axis_index("c")
  sid = lax.axis_index("s")
  wid = cid * NS + sid

  z = _zero16()
  for i in range(40):
    zv[pl.ds(i * 16, 16)] = z

  @pl.when(sid < 15)
  def _():
    pltpu.sync_copy(zv, deg_sh.at[pl.ds(sid * 640, 640)])

  @pl.when(sid == 15)
  def _():
    pltpu.sync_copy(zv.at[pl.ds(0, 400)], deg_sh.at[pl.ds(9600, 400)])

  plsc.subcore_barrier()

  base = wid * NBLK_DEG
  pltpu.sync_copy(c_hbm.at[pl.ds(base, NBLK_DEG)], c_v)
  pltpu.sync_copy(ew_hbm.at[pl.ds(base, NBLK_DEG)], ew_v)

  def blk(b, carry):
    pltpu.sync_copy(ew_v.at[b], deg_sh.at[c_v.at[b]], add=True)
    return carry

  lax.fori_loop(0, NBLK_DEG, blk, 0)
  plsc.subcore_barrier()

  def wb(dst):
    @pl.when(sid < 15)
    def _():
      pltpu.sync_copy(deg_sh.at[pl.ds(sid * 640, 640)], zv)
      pltpu.sync_copy(zv, dst.at[pl.ds(sid * 640, 640)])

    @pl.when(sid == 15)
    def _():
      pltpu.sync_copy(deg_sh.at[pl.ds(9600, 400)], zv.at[pl.ds(0, 400)])
      pltpu.sync_copy(zv.at[pl.ds(0, 400)], dst.at[pl.ds(9600, 400)])

  @pl.when(cid == 0)
  def _():
    wb(deg0_hbm)

  @pl.when(cid == 1)
  def _():
    wb(deg1_hbm)


def _make_deg_call(mesh):
  return pl.kernel(
      _deg_body,
      out_type=(
          jax.ShapeDtypeStruct((N,), jnp.float32),
          jax.ShapeDtypeStruct((N,), jnp.float32),
      ),
      mesh=mesh,
      scratch_types=[
          pltpu.VMEM((NBLK_DEG, B), jnp.int32),
          pltpu.VMEM((NBLK_DEG, B), jnp.float32),
          pltpu.VMEM((640,), jnp.float32),
          pltpu.VMEM_SHARED((N,), jnp.float32),
      ],
  )


def _prop_body(wh, g_lo, g_hi, r_hbm, c_hbm, ew_hbm, out_lo, out_hi,
               c_all, rv0, rv1, ev0, ev1, rows0, rows1,
               gs0, gs1, ss0, ss1, ms0, ms1, acc_sh):
  cid = lax.axis_index("c")
  sid = lax.axis_index("s")
  nseg = wh // 16

  _acc_init(acc_sh, rows0, sid, nseg)
  plsc.subcore_barrier()

  pltpu.sync_copy(c_hbm.at[pl.ds(sid * NBLK_TILE, NBLK_TILE)], c_all)

  def run(g_ref):
    _pipelined_edges(g_ref, acc_sh, r_hbm, ew_hbm, c_all, sid * NBLK_TILE,
                     rv0, rv1, ev0, ev1, rows0, rows1,
                     gs0, gs1, ss0, ss1, ms0, ms1, NBLK_TILE, nseg)

  @pl.when(cid == 0)
  def _():
    run(g_lo)

  @pl.when(cid == 1)
  def _():
    run(g_hi)

  plsc.subcore_barrier()

  @pl.when(cid == 0)
  def _():
    _acc_writeback(acc_sh, out_lo, sid)

  @pl.when(cid == 1)
  def _():
    _acc_writeback(acc_sh, out_hi, sid)


def _acc_init(acc_sh, rows_v, sid, nseg):
  z = _zero16()

  def zr(j, carry):
    for k in range(nseg):
      rows_v[j, pl.ds(k * 16, 16)] = z
    return carry

  lax.fori_loop(0, B, zr, 0)
  base_row = sid * 640

  @pl.when(sid < 15)
  def _():
    for kk in range(5):
      pltpu.sync_copy(rows_v, acc_sh.at[pl.ds(base_row + kk * B, B)])

  @pl.when(sid == 15)
  def _():
    for kk in range(3):
      pltpu.sync_copy(rows_v, acc_sh.at[pl.ds(base_row + kk * B, B)])
    pltpu.sync_copy(rows_v.at[pl.ds(0, 16)], acc_sh.at[pl.ds(9984, 16)])


def _acc_writeback(acc_sh, dst, sid):
  base_row = sid * 640

  @pl.when(sid < 15)
  def _():
    pltpu.sync_copy(acc_sh.at[pl.ds(base_row, 640)],
                    dst.at[pl.ds(base_row, 640)])

  @pl.when(sid == 15)
  def _():
    pltpu.sync_copy(acc_sh.at[pl.ds(base_row, 400)],
                    dst.at[pl.ds(base_row, 400)])


def _scale_block(rows_ref, ev, nseg):
  def grp(jj, carry):
    wv = ev[pl.ds(jj * 16, 16)]
    for l in range(16):
      j = jj * 16 + l
      w = _bcast_lane(wv, l)
      for k in range(nseg):
        rows_ref[j, pl.ds(k * 16, 16)] = rows_ref[j, pl.ds(k * 16, 16)] * w
    return carry

  lax.fori_loop(0, B // 16, grp, 0)


def _pipelined_edges(g_ref, acc_sh, r_hbm, ew_hbm, c_all, blkbase,
                     rv0, rv1, ev0, ev1, rows0, rows1,
                     gs0, gs1, ss0, ss1, ms0, ms1, nblk, nseg):
  def meta_start(b, rv, ev, ms):
    pltpu.async_copy(r_hbm.at[blkbase + b], rv, ms)
    pltpu.async_copy(ew_hbm.at[pl.ds((blkbase + b) * B, B)], ev, ms)

  def meta_wait(b, rv, ev, ms):
    pltpu.make_async_copy(r_hbm.at[blkbase + b], rv, ms).wait()
    pltpu.make_async_copy(ew_hbm.at[pl.ds((blkbase + b) * B, B)], ev, ms).wait()

  pltpu.sync_copy(r_hbm.at[blkbase], rv0)
  pltpu.sync_copy(ew_hbm.at[pl.ds(blkbase * B, B)], ev0)
  pltpu.async_copy(g_ref.at[rv0], rows0, gs0)
  meta_start(1, rv1, ev1, ms1)

  npair = nblk // 2

  def pair(p, carry):
    b0 = 2 * p
    b1 = b0 + 1
    pltpu.make_async_copy(g_ref.at[rv0], rows0, gs0).wait()

    @pl.when(p > 0)
    def _():
      pltpu.make_async_copy(rows1, acc_sh.at[c_all.at[b0 - 1]], ss1).wait()

    meta_wait(b1, rv1, ev1, ms1)
    pltpu.async_copy(g_ref.at[rv1], rows1, gs1)
    _scale_block(rows0, ev0, nseg)
    sd0 = pltpu.async_copy(rows0, acc_sh.at[c_all.at[b0]], ss0, add=True)

    @pl.when(b0 + 2 < nblk)
    def _():
      meta_start(b0 + 2, rv0, ev0, ms0)

    pltpu.make_async_copy(g_ref.at[rv1], rows1, gs1).wait()
    sd0.wait()

    @pl.when(p < npair - 1)
    def _():
      meta_wait(b0 + 2, rv0, ev0, ms0)
      pltpu.async_copy(g_ref.at[rv0], rows0, gs0)

    _scale_block(rows1, ev1, nseg)
    pltpu.async_copy(rows1, acc_sh.at[c_all.at[b1]], ss1, add=True)

    @pl.when(b1 + 2 < nblk)
    def _():
      meta_start(b1 + 2, rv1, ev1, ms1)

    return carry

  lax.fori_loop(0, npair, pair, 0)
  pltpu.make_async_copy(rows1, acc_sh.at[c_all.at[nblk - 1]], ss1).wait()


def _prop_split_body(g_hbm, r_hbm, c_hbm, ew_hbm, out0, out1,
                     c_all, rv0, rv1, ev0, ev1, rows0, rows1,
                     gs0, gs1, ss0, ss1, ms0, ms1, acc_sh):
  cid = lax.axis_index("c")
  sid = lax.axis_index("s")
  nseg = NCLS // 16
  nblk = NBLK_DEG

  _acc_init(acc_sh, rows0, sid, nseg)
  plsc.subcore_barrier()

  wid = cid * NS + sid
  pltpu.sync_copy(c_hbm.at[pl.ds(wid * nblk, nblk)], c_all)

  _pipelined_edges(g_hbm, acc_sh, r_hbm, ew_hbm, c_all, wid * nblk,
                   rv0, rv1, ev0, ev1, rows0, rows1,
                   gs0, gs1, ss0, ss1, ms0, ms1, nblk, nseg)
  plsc.subcore_barrier()

  @pl.when(cid == 0)
  def _():
    _acc_writeback(acc_sh, out0, sid)

  @pl.when(cid == 1)
  def _():
    _acc_writeback(acc_sh, out1, sid)


def _make_prop_split_call(mesh):
  return pl.kernel(
      _prop_split_body,
      out_type=(
          jax.ShapeDtypeStruct((N, NCLS), jnp.float32),
          jax.ShapeDtypeStruct((N, NCLS), jnp.float32),
      ),
      mesh=mesh,
      scratch_types=[
          pltpu.VMEM((NBLK_DEG, B), jnp.int32),
          pltpu.VMEM((B,), jnp.int32),
          pltpu.VMEM((B,), jnp.int32),
          pltpu.VMEM((B,), jnp.float32),
          pltpu.VMEM((B,), jnp.float32),
          pltpu.VMEM((B, NCLS), jnp.float32),
          pltpu.VMEM((B, NCLS), jnp.float32),
          pltpu.SemaphoreType.DMA,
          pltpu.SemaphoreType.DMA,
          pltpu.SemaphoreType.DMA,
          pltpu.SemaphoreType.DMA,
          pltpu.SemaphoreType.DMA,
          pltpu.SemaphoreType.DMA,
          pltpu.VMEM_SHARED((N, NCLS), jnp.float32),
      ],
  )


def _make_prop_call(mesh, wh):
  return pl.kernel(
      functools.partial(_prop_body, wh),
      out_type=(
          jax.ShapeDtypeStruct((N, wh), jnp.float32),
          jax.ShapeDtypeStruct((N, wh), jnp.float32),
      ),
      mesh=mesh,
      scratch_types=[
          pltpu.VMEM((NBLK_TILE, B), jnp.int32),
          pltpu.VMEM((B,), jnp.int32),
          pltpu.VMEM((B,), jnp.int32),
          pltpu.VMEM((B,), jnp.float32),
          pltpu.VMEM((B,), jnp.float32),
          pltpu.VMEM((B, wh), jnp.float32),
          pltpu.VMEM((B, wh), jnp.float32),
          pltpu.SemaphoreType.DMA,
          pltpu.SemaphoreType.DMA,
          pltpu.SemaphoreType.DMA,
          pltpu.SemaphoreType.DMA,
          pltpu.SemaphoreType.DMA,
          pltpu.SemaphoreType.DMA,
          pltpu.VMEM_SHARED((N, wh), jnp.float32),
      ],
  )


_GRID = 5
_RB = N // _GRID


def _row_spec(cols):
  return pl.BlockSpec((_RB, cols), lambda i: (i, 0))


def _full_spec(r, cols):
  return pl.BlockSpec((r, cols), lambda i: (0, 0))


def _dis(deg0_ref, deg1_ref):
  return lax.rsqrt(deg0_ref[...] + deg1_ref[...] + 1.0)


def _k1_body(x_ref, py_ref, lw_ref, lb_ref, w1_ref, d0_ref, d1_ref,
             glo_ref, ghi_ref):
  t = x_ref[...] + jnp.dot(py_ref[...], lw_ref[...],
                           preferred_element_type=jnp.float32) + lb_ref[...]
  g = _dis(d0_ref, d1_ref) * jnp.dot(t, w1_ref[...],
                                     preferred_element_type=jnp.float32)
  glo_ref[...] = g[:, :128]
  ghi_ref[...] = g[:, 128:]


def _k2_body(alo_ref, ahi_ref, glo_ref, ghi_ref, d0_ref, d1_ref, b_ref,
             w_ref, olo_ref, ohi_ref):
  dis = _dis(d0_ref, d1_ref)
  acc = jnp.concatenate([alo_ref[...], ahi_ref[...]], axis=1)
  g = jnp.concatenate([glo_ref[...], ghi_ref[...]], axis=1)
  h = jax.nn.relu(dis * (acc + g) + b_ref[...])
  g2 = dis * jnp.dot(h, w_ref[...], preferred_element_type=jnp.float32)
  olo_ref[...] = g2[:, :128]
  ohi_ref[...] = g2[:, 128:]


def _k3_body(alo_ref, ahi_ref, glo_ref, ghi_ref, d0_ref, d1_ref, b_ref,
             w_ref, f_ref, g3_ref):
  dis = _dis(d0_ref, d1_ref)
  acc = jnp.concatenate([alo_ref[...], ahi_ref[...]], axis=1)
  g = jnp.concatenate([glo_ref[...], ghi_ref[...]], axis=1)
  f = dis * (acc + g) + b_ref[...]
  f_ref[...] = f
  h2 = jax.nn.relu(f)
  g3_ref[...] = dis * jnp.dot(h2, w_ref[...],
                              preferred_element_type=jnp.float32)


def _k4_body(a0_ref, a1_ref, g_ref, d0_ref, d1_ref, b_ref, out_ref):
  dis = _dis(d0_ref, d1_ref)
  acc = a0_ref[...] + a1_ref[...]
  out_ref[...] = dis * (acc + g_ref[...]) + b_ref[...]


def kernel(x, pseudo_y, edge_index, egde_values, lin_W, lin_b,
           W1, b1, W2, b2, W3, b3):
  i32 = jnp.int32
  r = edge_index[0].astype(i32)
  c = edge_index[1].astype(i32)
  ew = egde_values.astype(jnp.float32)

  pad = E_PAD - E
  fill = (jnp.arange(pad, dtype=i32) * 13) % N
  r_p = jnp.concatenate([r, fill])
  c_p = jnp.concatenate([c, fill])
  ew_p = jnp.concatenate([ew, jnp.zeros((pad,), jnp.float32)])
  r2d = r_p.reshape(E_PAD // B, B)
  c2d = c_p.reshape(E_PAD // B, B)

  mesh = plsc.VectorSubcoreMesh(core_axis_name="c", subcore_axis_name="s")

  deg0, deg1 = _make_deg_call(mesh)(c2d, ew_p.reshape(E_PAD // B, B))
  d0 = deg0.reshape(N, 1)
  d1 = deg1.reshape(N, 1)

  dspec = pl.BlockSpec((_RB, 1), lambda i: (i, 0))
  lb2 = lin_b.reshape(1, D)
  b1_2 = b1.reshape(1, D)
  b2_2 = b2.reshape(1, D)
  b3_2 = b3.reshape(1, NCLS)

  g1_lo, g1_hi = pl.pallas_call(
      _k1_body,
      grid=(_GRID,),
      in_specs=[_row_spec(D), _row_spec(10), _full_spec(10, D),
                _full_spec(1, D), _full_spec(D, D), dspec, dspec],
      out_specs=(_row_spec(128), _row_spec(128)),
      out_shape=(jax.ShapeDtypeStruct((N, 128), jnp.float32),
                 jax.ShapeDtypeStruct((N, 128), jnp.float32)),
  )(x, pseudo_y, lin_W, lb2, W1, d0, d1)

  prop256 = _make_prop_call(mesh, 128)
  acc1_lo, acc1_hi = prop256(g1_lo, g1_hi, r2d, c2d, ew_p)

  g2_lo, g2_hi = pl.pallas_call(
      _k2_body,
      grid=(_GRID,),
      in_specs=[_row_spec(128), _row_spec(128), _row_spec(128), _row_spec(128),
                dspec, dspec, _full_spec(1, D), _full_spec(D, D)],
      out_specs=(_row_spec(128), _row_spec(128)),
      out_shape=(jax.ShapeDtypeStruct((N, 128), jnp.float32),
                 jax.ShapeDtypeStruct((N, 128), jnp.float32)),
  )(acc1_lo, acc1_hi, g1_lo, g1_hi, d0, d1, b1_2, W2)

  acc2_lo, acc2_hi = prop256(g2_lo, g2_hi, r2d, c2d, ew_p)

  f, g3 = pl.pallas_call(
      _k3_body,
      grid=(_GRID,),
      in_specs=[_row_spec(128), _row_spec(128), _row_spec(128), _row_spec(128),
                dspec, dspec, _full_spec(1, D), _full_spec(D, NCLS)],
      out_specs=(_row_spec(D), _row_spec(NCLS)),
      out_shape=(jax.ShapeDtypeStruct((N, D), jnp.float32),
                 jax.ShapeDtypeStruct((N, NCLS), jnp.float32)),
  )(acc2_lo, acc2_hi, g2_lo, g2_hi, d0, d1, b2_2, W3)

  prop3 = _make_prop_split_call(mesh)
  acc3_0, acc3_1 = prop3(g3, r2d, c2d, ew_p)

  out = pl.pallas_call(
      _k4_body,
      grid=(_GRID,),
      in_specs=[_row_spec(NCLS), _row_spec(NCLS), _row_spec(NCLS),
                dspec, dspec, _full_spec(1, NCLS)],
      out_specs=_row_spec(NCLS),
      out_shape=jax.ShapeDtypeStruct((N, NCLS), jnp.float32),
  )(acc3_0, acc3_1, g3, d0, d1, b3_2)

  return (f, out)


# speedup vs baseline: 1.0293x; 1.0128x over previous
import functools

import jax
import jax.numpy as jnp
from jax import lax
from jax.experimental import pallas as pl
from jax.experimental.pallas import tpu as pltpu
from jax.experimental.pallas import tpu_sc as plsc

N = 10000
E = 160000
D = 256
NCLS = 128

NC = 2
NS = 16
B = 128
E_PAD = 163840
NBLK_DEG = E_PAD // (NC * NS * B)
PB = 64
NBT = E_PAD // (NS * PB)
NBW = E_PAD // (NC * NS * PB)


def _zero16():
  return jnp.zeros((16,), jnp.float32)


def _bcast_lane(wv, l):
  idx = jnp.full((16, 1), l, jnp.int32)
  dn = lax.GatherDimensionNumbers(
      offset_dims=(), collapsed_slice_dims=(0,), start_index_map=(0,))
  return lax.gather(wv, idx, dn, slice_sizes=(1,),
                    mode=lax.GatherScatterMode.PROMISE_IN_BOUNDS)


def _deg_body(c_hbm, ew_hbm, deg0_hbm, deg1_hbm, c_v, ew_v, zv, deg_sh):
  cid = lax.axis_index("c")
  sid = lax.axis_index("s")
  wid = cid * NS + sid

  z = _zero16()
  for i in range(40):
    zv[pl.ds(i * 16, 16)] = z

  @pl.when(sid < 15)
  def _():
    pltpu.sync_copy(zv, deg_sh.at[pl.ds(sid * 640, 640)])

  @pl.when(sid == 15)
  def _():
    pltpu.sync_copy(zv.at[pl.ds(0, 400)], deg_sh.at[pl.ds(9600, 400)])

  plsc.subcore_barrier()

  base = wid * NBLK_DEG
  pltpu.sync_copy(c_hbm.at[pl.ds(base, NBLK_DEG)], c_v)
  pltpu.sync_copy(ew_hbm.at[pl.ds(base, NBLK_DEG)], ew_v)

  def blk(b, carry):
    pltpu.sync_copy(ew_v.at[b], deg_sh.at[c_v.at[b]], add=True)
    return carry

  lax.fori_loop(0, NBLK_DEG, blk, 0)
  plsc.subcore_barrier()

  def wb(dst):
    @pl.when(sid < 15)
    def _():
      pltpu.sync_copy(deg_sh.at[pl.ds(sid * 640, 640)], zv)
      pltpu.sync_copy(zv, dst.at[pl.ds(sid * 640, 640)])

    @pl.when(sid == 15)
    def _():
      pltpu.sync_copy(deg_sh.at[pl.ds(9600, 400)], zv.at[pl.ds(0, 400)])
      pltpu.sync_copy(zv.at[pl.ds(0, 400)], dst.at[pl.ds(9600, 400)])

  @pl.when(cid == 0)
  def _():
    wb(deg0_hbm)

  @pl.when(cid == 1)
  def _():
    wb(deg1_hbm)


def _make_deg_call(mesh):
  return pl.kernel(
      _deg_body,
      out_type=(
          jax.ShapeDtypeStruct((N,), jnp.float32),
          jax.ShapeDtypeStruct((N,), jnp.float32),
      ),
      mesh=mesh,
      scratch_types=[
          pltpu.VMEM((NBLK_DEG, B), jnp.int32),
          pltpu.VMEM((NBLK_DEG, B), jnp.float32),
          pltpu.VMEM((640,), jnp.float32),
          pltpu.VMEM_SHARED((N,), jnp.float32),
      ],
  )


NSLOT = 5


def _acc_init(acc_sh, rows_v, sid, nseg):
  z = _zero16()

  def zr(j, carry):
    for k in range(nseg):
      rows_v[j, pl.ds(k * 16, 16)] = z
    return carry

  lax.fori_loop(0, PB, zr, 0)
  base_row = sid * 640

  @pl.when(sid < 15)
  def _():
    for kk in range(10):
      pltpu.sync_copy(rows_v, acc_sh.at[pl.ds(base_row + kk * PB, PB)])

  @pl.when(sid == 15)
  def _():
    for kk in range(6):
      pltpu.sync_copy(rows_v, acc_sh.at[pl.ds(base_row + kk * PB, PB)])
    pltpu.sync_copy(rows_v.at[pl.ds(0, 16)], acc_sh.at[pl.ds(9984, 16)])


def _acc_writeback(acc_sh, dst, sid):
  base_row = sid * 640

  @pl.when(sid < 15)
  def _():
    pltpu.sync_copy(acc_sh.at[pl.ds(base_row, 640)],
                    dst.at[pl.ds(base_row, 640)])

  @pl.when(sid == 15)
  def _():
    pltpu.sync_copy(acc_sh.at[pl.ds(base_row, 400)],
                    dst.at[pl.ds(base_row, 400)])


def _scale_block(rows_ref, ev, nseg):
  def grp(jj, carry):
    wv = ev[pl.ds(jj * 16, 16)]
    for l in range(16):
      j = jj * 16 + l
      w = _bcast_lane(wv, l)
      for k in range(nseg):
        rows_ref[j, pl.ds(k * 16, 16)] = rows_ref[j, pl.ds(k * 16, 16)] * w
    return carry

  lax.fori_loop(0, PB // 16, grp, 0)


def _pipe5(g_ref, acc_sh, r_hbm, c_hbm, ew_hbm, blkbase,
           rows, rvs, cvs, evs, gss, sss, mss, nblk, nseg):
  assert nblk % NSLOT == 0 and nblk >= 2 * NSLOT

  def meta_start(b, q):
    pltpu.async_copy(r_hbm.at[blkbase + b], rvs[q], mss[q])
    pltpu.async_copy(c_hbm.at[blkbase + b], cvs[q], mss[q])
    pltpu.async_copy(ew_hbm.at[pl.ds((blkbase + b) * PB, PB)], evs[q], mss[q])

  def meta_wait(b, q):
    pltpu.make_async_copy(r_hbm.at[blkbase + b], rvs[q], mss[q]).wait()
    pltpu.make_async_copy(c_hbm.at[blkbase + b], cvs[q], mss[q]).wait()
    pltpu.make_async_copy(
        ew_hbm.at[pl.ds((blkbase + b) * PB, PB)], evs[q], mss[q]).wait()

  def gather_start(q):
    pltpu.async_copy(g_ref.at[rvs[q]], rows[q], gss[q])

  def gather_wait(q):
    pltpu.make_async_copy(g_ref.at[rvs[q]], rows[q], gss[q]).wait()

  def scat_start(q):
    pltpu.async_copy(rows[q], acc_sh.at[cvs[q]], sss[q], add=True)

  def scat_wait(q):
    pltpu.make_async_copy(rows[q], acc_sh.at[cvs[q]], sss[q]).wait()

  pltpu.sync_copy(r_hbm.at[blkbase], rvs[0])
  pltpu.sync_copy(c_hbm.at[blkbase], cvs[0])
  pltpu.sync_copy(ew_hbm.at[pl.ds(blkbase * PB, PB)], evs[0])
  gather_start(0)
  meta_start(1, 1)
  meta_start(2, 2)
  meta_wait(1, 1)
  gather_start(1)
  gather_wait(0)
  _scale_block(rows[0], evs[0], nseg)
  scat_start(0)
  meta_start(3, 3)
  meta_wait(2, 2)
  gather_start(2)
  gather_wait(1)
  _scale_block(rows[1], evs[1], nseg)
  scat_start(1)
  meta_start(4, 4)
  meta_wait(3, 3)
  gather_start(3)

  def group(i, carry):
    base_b = 2 + i * NSLOT
    for j in range(NSLOT):
      b = base_b + j
      q = (2 + j) % NSLOT
      gather_wait(q)
      _scale_block(rows[q], evs[q], nseg)
      scat_start(q)
      scat_wait((q + 3) % NSLOT)
      meta_start(b + 3, (q + 3) % NSLOT)
      meta_wait(b + 2, (q + 2) % NSLOT)
      gather_start((q + 2) % NSLOT)
    return carry

  lax.fori_loop(0, (nblk - NSLOT) // NSLOT, group, 0)

  b = nblk - 3
  q = b % NSLOT
  gather_wait(q)
  _scale_block(rows[q], evs[q], nseg)
  scat_start(q)
  scat_wait((q + 3) % NSLOT)
  meta_wait(nblk - 1, (q + 2) % NSLOT)
  gather_start((q + 2) % NSLOT)
  b = nblk - 2
  q = b % NSLOT
  gather_wait(q)
  _scale_block(rows[q], evs[q], nseg)
  scat_start(q)
  scat_wait((q + 3) % NSLOT)
  b = nblk - 1
  q = b % NSLOT
  gather_wait(q)
  _scale_block(rows[q], evs[q], nseg)
  scat_start(q)
  scat_wait((q + 3) % NSLOT)
  scat_wait((nblk - 2) % NSLOT)
  scat_wait((nblk - 1) % NSLOT)


def _prop_body(g_lo, g_hi, r_hbm, c_hbm, ew_hbm, out_lo, out_hi,
               rows, rvs, cvs, evs, gss, sss, mss, acc_sh):
  cid = lax.axis_index("c")
  sid = lax.axis_index("s")
  nseg = 8

  _acc_init(acc_sh, rows[0], sid, nseg)
  plsc.subcore_barrier()

  def run(g_ref):
    _pipe5(g_ref, acc_sh, r_hbm, c_hbm, ew_hbm, sid * NBT,
           rows, rvs, cvs, evs, gss, sss, mss, NBT, nseg)

  @pl.when(cid == 0)
  def _():
    run(g_lo)

  @pl.when(cid == 1)
  def _():
    run(g_hi)

  plsc.subcore_barrier()

  @pl.when(cid == 0)
  def _():
    _acc_writeback(acc_sh, out_lo, sid)

  @pl.when(cid == 1)
  def _():
    _acc_writeback(acc_sh, out_hi, sid)


def _prop_split_body(g_hbm, r_hbm, c_hbm, ew_hbm, out0, out1,
                     rows, rvs, cvs, evs, gss, sss, mss, acc_sh):
  cid = lax.axis_index("c")
  sid = lax.axis_index("s")
  nseg = 8

  _acc_init(acc_sh, rows[0], sid, nseg)
  plsc.subcore_barrier()

  wid = cid * NS + sid
  _pipe5(g_hbm, acc_sh, r_hbm, c_hbm, ew_hbm, wid * NBW,
         rows, rvs, cvs, evs, gss, sss, mss, NBW, nseg)

  plsc.subcore_barrier()

  @pl.when(cid == 0)
  def _():
    _acc_writeback(acc_sh, out0, sid)

  @pl.when(cid == 1)
  def _():
    _acc_writeback(acc_sh, out1, sid)


def _prop_scratch(wh):
  return [
      [pltpu.VMEM((PB, wh), jnp.float32) for _ in range(NSLOT)],
      [pltpu.VMEM((PB,), jnp.int32) for _ in range(NSLOT)],
      [pltpu.VMEM((PB,), jnp.int32) for _ in range(NSLOT)],
      [pltpu.VMEM((PB,), jnp.float32) for _ in range(NSLOT)],
      [pltpu.SemaphoreType.DMA for _ in range(NSLOT)],
      [pltpu.SemaphoreType.DMA for _ in range(NSLOT)],
      [pltpu.SemaphoreType.DMA for _ in range(NSLOT)],
      pltpu.VMEM_SHARED((N, wh), jnp.float32),
  ]


def _make_prop_call(mesh, wh):
  return pl.kernel(
      _prop_body,
      out_type=(
          jax.ShapeDtypeStruct((N, wh), jnp.float32),
          jax.ShapeDtypeStruct((N, wh), jnp.float32),
      ),
      mesh=mesh,
      scratch_types=_prop_scratch(wh),
  )


def _make_prop_split_call(mesh):
  return pl.kernel(
      _prop_split_body,
      out_type=(
          jax.ShapeDtypeStruct((N, NCLS), jnp.float32),
          jax.ShapeDtypeStruct((N, NCLS), jnp.float32),
      ),
      mesh=mesh,
      scratch_types=_prop_scratch(NCLS),
  )


_GRID = 5
_RB = N // _GRID


def _row_spec(cols):
  return pl.BlockSpec((_RB, cols), lambda i: (i, 0))


def _full_spec(r, cols):
  return pl.BlockSpec((r, cols), lambda i: (0, 0))


def _dis(deg0_ref, deg1_ref):
  return lax.rsqrt(deg0_ref[...] + deg1_ref[...] + 1.0)


def _k1_body(x_ref, py_ref, lw_ref, lb_ref, w1_ref, d0_ref, d1_ref,
             glo_ref, ghi_ref):
  t = x_ref[...] + jnp.dot(py_ref[...], lw_ref[...],
                           preferred_element_type=jnp.float32) + lb_ref[...]
  g = _dis(d0_ref, d1_ref) * jnp.dot(t, w1_ref[...],
                                     preferred_element_type=jnp.float32)
  glo_ref[...] = g[:, :128]
  ghi_ref[...] = g[:, 128:]


def _k2_body(alo_ref, ahi_ref, glo_ref, ghi_ref, d0_ref, d1_ref, b_ref,
             w_ref, olo_ref, ohi_ref):
  dis = _dis(d0_ref, d1_ref)
  acc = jnp.concatenate([alo_ref[...], ahi_ref[...]], axis=1)
  g = jnp.concatenate([glo_ref[...], ghi_ref[...]], axis=1)
  h = jax.nn.relu(dis * (acc + g) + b_ref[...])
  g2 = dis * jnp.dot(h, w_ref[...], preferred_element_type=jnp.float32)
  olo_ref[...] = g2[:, :128]
  ohi_ref[...] = g2[:, 128:]


def _k3_body(alo_ref, ahi_ref, glo_ref, ghi_ref, d0_ref, d1_ref, b_ref,
             w_ref, f_ref, g3_ref):
  dis = _dis(d0_ref, d1_ref)
  acc = jnp.concatenate([alo_ref[...], ahi_ref[...]], axis=1)
  g = jnp.concatenate([glo_ref[...], ghi_ref[...]], axis=1)
  f = dis * (acc + g) + b_ref[...]
  f_ref[...] = f
  h2 = jax.nn.relu(f)
  g3_ref[...] = dis * jnp.dot(h2, w_ref[...],
                              preferred_element_type=jnp.float32)


def _k4_body(a0_ref, a1_ref, g_ref, d0_ref, d1_ref, b_ref, out_ref):
  dis = _dis(d0_ref, d1_ref)
  acc = a0_ref[...] + a1_ref[...]
  out_ref[...] = dis * (acc + g_ref[...]) + b_ref[...]


def kernel(x, pseudo_y, edge_index, egde_values, lin_W, lin_b,
           W1, b1, W2, b2, W3, b3):
  i32 = jnp.int32
  r = edge_index[0].astype(i32)
  c = edge_index[1].astype(i32)
  ew = egde_values.astype(jnp.float32)

  pad = E_PAD - E
  fill = (jnp.arange(pad, dtype=i32) * 13) % N
  r_p = jnp.concatenate([r, fill])
  c_p = jnp.concatenate([c, fill])
  ew_p = jnp.concatenate([ew, jnp.zeros((pad,), jnp.float32)])
  r2d = r_p.reshape(E_PAD // PB, PB)
  c2d = c_p.reshape(E_PAD // PB, PB)

  mesh = plsc.VectorSubcoreMesh(core_axis_name="c", subcore_axis_name="s")

  deg0, deg1 = _make_deg_call(mesh)(
      c_p.reshape(E_PAD // B, B), ew_p.reshape(E_PAD // B, B))
  d0 = deg0.reshape(N, 1)
  d1 = deg1.reshape(N, 1)

  dspec = pl.BlockSpec((_RB, 1), lambda i: (i, 0))
  lb2 = lin_b.reshape(1, D)
  b1_2 = b1.reshape(1, D)
  b2_2 = b2.reshape(1, D)
  b3_2 = b3.reshape(1, NCLS)

  g1_lo, g1_hi = pl.pallas_call(
      _k1_body,
      grid=(_GRID,),
      in_specs=[_row_spec(D), _row_spec(10), _full_spec(10, D),
                _full_spec(1, D), _full_spec(D, D), dspec, dspec],
      out_specs=(_row_spec(128), _row_spec(128)),
      out_shape=(jax.ShapeDtypeStruct((N, 128), jnp.float32),
                 jax.ShapeDtypeStruct((N, 128), jnp.float32)),
  )(x, pseudo_y, lin_W, lb2, W1, d0, d1)

  prop256 = _make_prop_call(mesh, 128)
  acc1_lo, acc1_hi = prop256(g1_lo, g1_hi, r2d, c2d, ew_p)

  g2_lo, g2_hi = pl.pallas_call(
      _k2_body,
      grid=(_GRID,),
      in_specs=[_row_spec(128), _row_spec(128), _row_spec(128), _row_spec(128),
                dspec, dspec, _full_spec(1, D), _full_spec(D, D)],
      out_specs=(_row_spec(128), _row_spec(128)),
      out_shape=(jax.ShapeDtypeStruct((N, 128), jnp.float32),
                 jax.ShapeDtypeStruct((N, 128), jnp.float32)),
  )(acc1_lo, acc1_hi, g1_lo, g1_hi, d0, d1, b1_2, W2)

  acc2_lo, acc2_hi = prop256(g2_lo, g2_hi, r2d, c2d, ew_p)

  f, g3 = pl.pallas_call(
      _k3_body,
      grid=(_GRID,),
      in_specs=[_row_spec(128), _row_spec(128), _row_spec(128), _row_spec(128),
                dspec, dspec, _full_spec(1, D), _full_spec(D, NCLS)],
      out_specs=(_row_spec(D), _row_spec(NCLS)),
      out_shape=(jax.ShapeDtypeStruct((N, D), jnp.float32),
                 jax.ShapeDtypeStruct((N, NCLS), jnp.float32)),
  )(acc2_lo, acc2_hi, g2_lo, g2_hi, d0, d1, b2_2, W3)

  prop3 = _make_prop_split_call(mesh)
  acc3_0, acc3_1 = prop3(g3, r2d, c2d, ew_p)

  out = pl.pallas_call(
      _k4_body,
      grid=(_GRID,),
      in_specs=[_row_spec(NCLS), _row_spec(NCLS), _row_spec(NCLS),
                dspec, dspec, _full_spec(1, NCLS)],
      out_specs=_row_spec(NCLS),
      out_shape=jax.ShapeDtypeStruct((N, NCLS), jnp.float32),
  )(acc3_0, acc3_1, g3, d0, d1, b3_2)

  return (f, out)
